# Initial kernel scaffold; baseline (speedup 1.0000x reference)
import jax
import jax.numpy as jnp
from jax.experimental import pallas as pl


def kernel(x, c, edge_index, W1_0, b1_0, Wc_0, bc_0, W2_0, b2_0, W1_1, b1_1, Wc_1, bc_1, W2_1, b2_1):
    raise NotImplementedError("write your pallas kernel here")


# speedup vs baseline: 4.1042x; 4.1042x over previous
import functools

import jax
import jax.numpy as jnp
from jax import lax
from jax.experimental import pallas as pl
from jax.experimental.pallas import tpu as pltpu
from jax.experimental.pallas import tpu_sc as plsc

_N = 50000
_E = 800000
_H = 64
_HH = 32
_CHUNK = 128
_TILES = 16
_CPT = 391
_EPAD = _TILES * _CPT * _CHUNK
_NP = 51200
_ZB = _NP // (_TILES * _CHUNK)
_ROWS_OUT = 3128
_ROWS_LAST = _N - 15 * _ROWS_OUT
_R = 2000


def _sc_aggregate(with_deg):
  mesh = plsc.VectorSubcoreMesh(core_axis_name="c", subcore_axis_name="s")
  out_type = [jax.ShapeDtypeStruct((2, _N, _HH), jnp.float32)]
  scratch = [
      pltpu.VMEM((_CHUNK,), jnp.int32),
      pltpu.VMEM((_CHUNK,), jnp.int32),
      pltpu.VMEM((_CHUNK, _HH), jnp.float32),
      pltpu.VMEM((_CHUNK, _HH), jnp.float32),
      pltpu.VMEM_SHARED((_NP, _HH), jnp.float32),
      pltpu.SemaphoreType.DMA,
  ]
  if with_deg:
    out_type.append(jax.ShapeDtypeStruct((_N,), jnp.float32))
    scratch += [
        pltpu.VMEM((_CHUNK,), jnp.float32),
        pltpu.VMEM((3200,), jnp.float32),
        pltpu.VMEM_SHARED((_NP,), jnp.float32),
    ]

  def body(h2, srcp, dstp, *refs):
    if with_deg:
      (agg_out, deg_out, src_v, dst_v, rows_v, zrow_v, agg_sh, sem,
       ones_v, dzero_v, deg_sh) = refs
    else:
      agg_out, src_v, dst_v, rows_v, zrow_v, agg_sh, sem = refs
    c = lax.axis_index("c")
    s = lax.axis_index("s")

    def _zrow(i, _):
      zrow_v[i, pl.ds(0, 16)] = jnp.zeros((16,), jnp.float32)
      zrow_v[i, pl.ds(16, 16)] = jnp.zeros((16,), jnp.float32)
      return 0
    lax.fori_loop(0, _CHUNK, _zrow, 0)

    def _zshared(j, _):
      pltpu.sync_copy(zrow_v, agg_sh.at[pl.ds((s * _ZB + j) * _CHUNK, _CHUNK)])
      return 0
    lax.fori_loop(0, _ZB, _zshared, 0)

    if with_deg:
      def _zd(i, _):
        dzero_v[pl.ds(i * 16, 16)] = jnp.zeros((16,), jnp.float32)
        return 0
      lax.fori_loop(0, 200, _zd, 0)
      pltpu.sync_copy(dzero_v, deg_sh.at[pl.ds(s * 3200, 3200)])

      def _ones(i, _):
        ones_v[pl.ds(i * 16, 16)] = jnp.ones((16,), jnp.float32)
        return 0
      lax.fori_loop(0, _CHUNK // 16, _ones, 0)

    plsc.subcore_barrier()

    bias = c * _N
    base = s * _CPT

    def step(i, _):
      off = (base + i) * _CHUNK
      pltpu.sync_copy(srcp.at[pl.ds(off, _CHUNK)], src_v)
      pltpu.sync_copy(dstp.at[pl.ds(off, _CHUNK)], dst_v)

      def _bias(j, _2):
        sl = pl.ds(j * 16, 16)
        src_v[sl] = src_v[sl] + bias
        return 0
      lax.fori_loop(0, _CHUNK // 16, _bias, 0)

      pltpu.async_copy(h2.at[src_v], rows_v, sem).wait()
      pltpu.sync_copy(rows_v, agg_sh.at[dst_v], add=True)
      if with_deg:
        @pl.when(c == 0)
        def _():
          pltpu.sync_copy(ones_v, deg_sh.at[dst_v], add=True)
      return 0
    lax.fori_loop(0, _CPT, step, 0)

    plsc.subcore_barrier()

    r0 = s * _ROWS_OUT

    @pl.when(s < 15)
    def _():
      pltpu.sync_copy(agg_sh.at[pl.ds(r0, _ROWS_OUT)],
                      agg_out.at[c, pl.ds(r0, _ROWS_OUT)])

    @pl.when(s == 15)
    def _():
      pltpu.sync_copy(agg_sh.at[pl.ds(15 * _ROWS_OUT, _ROWS_LAST)],
                      agg_out.at[c, pl.ds(15 * _ROWS_OUT, _ROWS_LAST)])
    if with_deg:
      @pl.when(jnp.logical_and(c == 0, s == 0))
      def _():
        pltpu.sync_copy(deg_sh.at[pl.ds(0, _N)], deg_out)

  return pl.kernel(
      body, out_type=out_type, mesh=mesh, scratch_types=scratch,
      compiler_params=pltpu.CompilerParams(use_tc_tiling_on_sc=False))


_sc_agg_deg = _sc_aggregate(True)
_sc_agg = _sc_aggregate(False)


def _full(shape):
  return pl.BlockSpec(shape, lambda i: tuple(0 for _ in shape))


def _tc_pre(x, cc, w1x, w1c, b1, wc, bc):
  def body(x_r, c_r, w1x_r, w1c_r, b1_r, wc_r, bc_r, hp_r, c1_r):
    h = jnp.maximum(
        jnp.dot(x_r[...], w1x_r[...], preferred_element_type=jnp.float32)
        + jnp.dot(c_r[...], w1c_r[...], preferred_element_type=jnp.float32)
        + b1_r[...], 0.0)
    c1 = jnp.maximum(
        jnp.dot(c_r[...], wc_r[...], preferred_element_type=jnp.float32)
        + bc_r[...], 0.0)
    hp_r[0] = h[:, :_HH]
    hp_r[1] = h[:, _HH:]
    c1_r[...] = c1

  return pl.pallas_call(
      body,
      grid=(_N // _R,),
      in_specs=[
          pl.BlockSpec((_R, 128), lambda i: (i, 0)),
          pl.BlockSpec((_R, 16), lambda i: (i, 0)),
          _full((128, _H)), _full((16, _H)), _full((1, _H)),
          _full((16, _H)), _full((1, _H)),
      ],
      out_specs=[
          pl.BlockSpec((2, _R, _HH), lambda i: (0, i, 0)),
          pl.BlockSpec((_R, _H), lambda i: (i, 0)),
      ],
      out_shape=[
          jax.ShapeDtypeStruct((2, _N, _HH), jnp.float32),
          jax.ShapeDtypeStruct((_N, _H), jnp.float32),
      ],
  )(x, cc, w1x, w1c, b1, wc, bc)


def _tc_mid(hp, agg, deg, c1, w2, b2, w1a, w1b, b11):
  def body(hlo, hhi, alo, ahi, deg_r, c1_r, w2_r, b2_r, w1a_r, w1b_r, b11_r,
           out_r):
    inv = 1.0 / jnp.maximum(deg_r[...], 1.0)
    u = jnp.concatenate(
        [hlo[0] + alo[0] * inv, hhi[0] + ahi[0] * inv], axis=1)
    x1 = jnp.dot(u, w2_r[...], preferred_element_type=jnp.float32) + b2_r[...]
    h1 = jnp.maximum(
        jnp.dot(x1, w1a_r[...], preferred_element_type=jnp.float32)
        + jnp.dot(c1_r[...], w1b_r[...], preferred_element_type=jnp.float32)
        + b11_r[...], 0.0)
    out_r[0] = h1[:, :_HH]
    out_r[1] = h1[:, _HH:]

  lo = pl.BlockSpec((1, _R, _HH), lambda i: (0, i, 0))
  hi = pl.BlockSpec((1, _R, _HH), lambda i: (1, i, 0))
  return pl.pallas_call(
      body,
      grid=(_N // _R,),
      in_specs=[
          lo, hi, lo, hi,
          pl.BlockSpec((_R, 1), lambda i: (i, 0)),
          pl.BlockSpec((_R, _H), lambda i: (i, 0)),
          _full((_H, _H)), _full((1, _H)),
          _full((_H, _H)), _full((_H, _H)), _full((1, _H)),
      ],
      out_specs=pl.BlockSpec((2, _R, _HH), lambda i: (0, i, 0)),
      out_shape=jax.ShapeDtypeStruct((2, _N, _HH), jnp.float32),
  )(hp, hp, agg, agg, deg, c1, w2, b2, w1a, w1b, b11)


def _tc_fin(hp, agg, deg, w2, b2):
  def body(hlo, hhi, alo, ahi, deg_r, w2_r, b2_r, out_r):
    inv = 1.0 / jnp.maximum(deg_r[...], 1.0)
    u = jnp.concatenate(
        [hlo[0] + alo[0] * inv, hhi[0] + ahi[0] * inv], axis=1)
    out_r[...] = (
        jnp.dot(u, w2_r[...], preferred_element_type=jnp.float32) + b2_r[...])

  lo = pl.BlockSpec((1, _R, _HH), lambda i: (0, i, 0))
  hi = pl.BlockSpec((1, _R, _HH), lambda i: (1, i, 0))
  return pl.pallas_call(
      body,
      grid=(_N // _R,),
      in_specs=[
          lo, hi, lo, hi,
          pl.BlockSpec((_R, 1), lambda i: (i, 0)),
          _full((_H, 128)), _full((1, 128)),
      ],
      out_specs=pl.BlockSpec((_R, 128), lambda i: (i, 0)),
      out_shape=jax.ShapeDtypeStruct((_N, 128), jnp.float32),
  )(hp, hp, agg, agg, deg, w2, b2)


def kernel(x, c, edge_index, W1_0, b1_0, Wc_0, bc_0, W2_0, b2_0,
           W1_1, b1_1, Wc_1, bc_1, W2_1, b2_1):
  src = edge_index[0]
  dst = edge_index[1]
  pad = _EPAD - _E
  srcp = jnp.concatenate([src, jnp.zeros((pad,), jnp.int32)])
  dstp = jnp.concatenate([dst, jnp.full((pad,), _N, jnp.int32)])

  hp0, c1 = _tc_pre(x, c, W1_0[:128], W1_0[128:],
                    b1_0.reshape(1, _H), Wc_0, bc_0.reshape(1, _H))
  agg0, deg = _sc_agg_deg(hp0.reshape(2 * _N, _HH), srcp, dstp)
  deg2 = deg.reshape(_N, 1)
  hp1 = _tc_mid(hp0, agg0, deg2, c1, W2_0, b2_0.reshape(1, _H),
                W1_1[:_H], W1_1[_H:], b1_1.reshape(1, _H))
  (agg1,) = _sc_agg(hp1.reshape(2 * _N, _HH), srcp, dstp)
  return _tc_fin(hp1, agg1, deg2, W2_1, b2_1.reshape(1, 128))


# speedup vs baseline: 10.5118x; 2.5612x over previous
import functools

import jax
import jax.numpy as jnp
from jax import lax
from jax.experimental import pallas as pl
from jax.experimental.pallas import tpu as pltpu
from jax.experimental.pallas import tpu_sc as plsc

_N = 50000
_E = 800000
_H = 64
_HH = 32
_CHUNK = 128
_TILES = 16
_CPT = 392
_EPAD = _TILES * _CPT * _CHUNK
_NROWS = _EPAD // _CHUNK
_NBUF = 4
_NP = 51200
_ZB = _NP // (_TILES * _CHUNK)
_ROWS_OUT = 3128
_ROWS_LAST = _N - 15 * _ROWS_OUT
_R = 2000


def _sc_aggregate():
  mesh = plsc.VectorSubcoreMesh(core_axis_name="c", subcore_axis_name="s")
  out_type = [jax.ShapeDtypeStruct((2, _N, _HH), jnp.float32),
              jax.ShapeDtypeStruct((_N,), jnp.float32)]
  scratch = [
      pltpu.VMEM((8, _CHUNK), jnp.int32),
      pltpu.VMEM((8, _CHUNK), jnp.int32),
      pltpu.VMEM((_NBUF, _CHUNK, _HH), jnp.float32),
      pltpu.VMEM_SHARED((_NP, _HH), jnp.float32),
      pltpu.SemaphoreType.DMA,
      pltpu.SemaphoreType.DMA,
      pltpu.SemaphoreType.DMA,
      pltpu.VMEM((_CHUNK,), jnp.float32),
      pltpu.VMEM((3200,), jnp.float32),
      pltpu.VMEM_SHARED((_NP,), jnp.float32),
  ]

  def body(h2, src3, dst2, agg_out, deg_out, src_v, dst_v, rows_v, agg_sh,
           sem_i, sem_g, sem_s, ones_v, dzero_v, deg_sh):
    c = lax.axis_index("c")
    s = lax.axis_index("s")

    def _zrow(i, _):
      rows_v[0, i, pl.ds(0, 16)] = jnp.zeros((16,), jnp.float32)
      rows_v[0, i, pl.ds(16, 16)] = jnp.zeros((16,), jnp.float32)
      return 0
    lax.fori_loop(0, _CHUNK, _zrow, 0)

    def _zshared(j, _):
      pltpu.sync_copy(rows_v.at[0],
                      agg_sh.at[pl.ds((s * _ZB + j) * _CHUNK, _CHUNK)])
      return 0
    lax.fori_loop(0, _ZB, _zshared, 0)

    def _zd(i, _):
      dzero_v[pl.ds(i * 16, 16)] = jnp.zeros((16,), jnp.float32)
      return 0
    lax.fori_loop(0, 200, _zd, 0)
    pltpu.sync_copy(dzero_v, deg_sh.at[pl.ds(s * 3200, 3200)])

    def _ones(i, _):
      ones_v[pl.ds(i * 16, 16)] = jnp.ones((16,), jnp.float32)
      return 0
    lax.fori_loop(0, _CHUNK // 16, _ones, 0)

    plsc.subcore_barrier()

    row0 = s * _CPT

    def issue_idx(j, b):
      pltpu.async_copy(src3.at[c, row0 + j], src_v.at[b], sem_i)
      pltpu.async_copy(dst2.at[row0 + j], dst_v.at[b], sem_i)

    def wait_idx_pair():
      pltpu.make_async_copy(dst2.at[0], src_v.at[0], sem_i).wait()
      pltpu.make_async_copy(dst2.at[0], dst_v.at[0], sem_i).wait()

    def wait_chunk(sem):
      pltpu.make_async_copy(h2.at[pl.ds(0, _CHUNK)], rows_v.at[0], sem).wait()

    for b in range(4):
      issue_idx(b, b)
    wait_idx_pair()
    wait_idx_pair()
    pltpu.async_copy(h2.at[src_v.at[0]], rows_v.at[0], sem_g)
    pltpu.async_copy(h2.at[src_v.at[1]], rows_v.at[1], sem_g)

    def grp(g, _):
      for b in range(8):
        j = g * 8 + b

        @pl.when(j + 4 < _CPT)
        def _():
          issue_idx(j + 4, (b + 4) % 8)

        @pl.when(j >= 2)
        def _():
          wait_chunk(sem_s)

        @pl.when(j + 2 < _CPT)
        def _():
          wait_idx_pair()
          pltpu.async_copy(h2.at[src_v.at[(b + 2) % 8]],
                           rows_v.at[(b + 2) % _NBUF], sem_g)

        wait_chunk(sem_g)
        pltpu.async_copy(rows_v.at[b % _NBUF], agg_sh.at[dst_v.at[b % 8]],
                         sem_s, add=True)

        @pl.when(c == 0)
        def _():
          pltpu.sync_copy(ones_v, deg_sh.at[dst_v.at[b % 8]], add=True)
      return 0
    lax.fori_loop(0, _CPT // 8, grp, 0)

    wait_chunk(sem_s)
    wait_chunk(sem_s)

    plsc.subcore_barrier()

    r0 = s * _ROWS_OUT

    @pl.when(s < 15)
    def _():
      pltpu.sync_copy(agg_sh.at[pl.ds(r0, _ROWS_OUT)],
                      agg_out.at[c, pl.ds(r0, _ROWS_OUT)])

    @pl.when(s == 15)
    def _():
      pltpu.sync_copy(agg_sh.at[pl.ds(15 * _ROWS_OUT, _ROWS_LAST)],
                      agg_out.at[c, pl.ds(15 * _ROWS_OUT, _ROWS_LAST)])

    @pl.when(jnp.logical_and(c == 0, s == 0))
    def _():
      pltpu.sync_copy(deg_sh.at[pl.ds(0, _N)], deg_out)

  return pl.kernel(
      body, out_type=out_type, mesh=mesh, scratch_types=scratch,
      compiler_params=pltpu.CompilerParams(use_tc_tiling_on_sc=False))


_sc_agg_deg = _sc_aggregate()


def _full(shape):
  return pl.BlockSpec(shape, lambda i: tuple(0 for _ in shape))


def _tc_pre(x, cc, w1x, w1c, b1, wc, bc):
  def body(x_r, c_r, w1x_r, w1c_r, b1_r, wc_r, bc_r, hp_r, c1_r):
    h = jnp.maximum(
        jnp.dot(x_r[...], w1x_r[...], preferred_element_type=jnp.float32)
        + jnp.dot(c_r[...], w1c_r[...], preferred_element_type=jnp.float32)
        + b1_r[...], 0.0)
    c1 = jnp.maximum(
        jnp.dot(c_r[...], wc_r[...], preferred_element_type=jnp.float32)
        + bc_r[...], 0.0)
    hp_r[0] = h[:, :_HH]
    hp_r[1] = h[:, _HH:]
    c1_r[...] = c1

  return pl.pallas_call(
      body,
      grid=(_N // _R,),
      in_specs=[
          pl.BlockSpec((_R, 128), lambda i: (i, 0)),
          pl.BlockSpec((_R, 16), lambda i: (i, 0)),
          _full((128, _H)), _full((16, _H)), _full((1, _H)),
          _full((16, _H)), _full((1, _H)),
      ],
      out_specs=[
          pl.BlockSpec((2, _R, _HH), lambda i: (0, i, 0)),
          pl.BlockSpec((_R, _H), lambda i: (i, 0)),
      ],
      out_shape=[
          jax.ShapeDtypeStruct((2, _N, _HH), jnp.float32),
          jax.ShapeDtypeStruct((_N, _H), jnp.float32),
      ],
  )(x, cc, w1x, w1c, b1, wc, bc)


def _tc_mid(hp, agg, deg, c1, w2, b2, w1a, w1b, b11):
  def body(hlo, hhi, alo, ahi, deg_r, c1_r, w2_r, b2_r, w1a_r, w1b_r, b11_r,
           out_r):
    inv = 1.0 / jnp.maximum(deg_r[...], 1.0)
    u = jnp.concatenate(
        [hlo[0] + alo[0] * inv, hhi[0] + ahi[0] * inv], axis=1)
    x1 = jnp.dot(u, w2_r[...], preferred_element_type=jnp.float32) + b2_r[...]
    h1 = jnp.maximum(
        jnp.dot(x1, w1a_r[...], preferred_element_type=jnp.float32)
        + jnp.dot(c1_r[...], w1b_r[...], preferred_element_type=jnp.float32)
        + b11_r[...], 0.0)
    out_r[0] = h1[:, :_HH]
    out_r[1] = h1[:, _HH:]

  lo = pl.BlockSpec((1, _R, _HH), lambda i: (0, i, 0))
  hi = pl.BlockSpec((1, _R, _HH), lambda i: (1, i, 0))
  return pl.pallas_call(
      body,
      grid=(_N // _R,),
      in_specs=[
          lo, hi, lo, hi,
          pl.BlockSpec((_R, 1), lambda i: (i, 0)),
          pl.BlockSpec((_R, _H), lambda i: (i, 0)),
          _full((_H, _H)), _full((1, _H)),
          _full((_H, _H)), _full((_H, _H)), _full((1, _H)),
      ],
      out_specs=pl.BlockSpec((2, _R, _HH), lambda i: (0, i, 0)),
      out_shape=jax.ShapeDtypeStruct((2, _N, _HH), jnp.float32),
  )(hp, hp, agg, agg, deg, c1, w2, b2, w1a, w1b, b11)


def _tc_fin(hp, agg, deg, w2, b2):
  def body(hlo, hhi, alo, ahi, deg_r, w2_r, b2_r, out_r):
    inv = 1.0 / jnp.maximum(deg_r[...], 1.0)
    u = jnp.concatenate(
        [hlo[0] + alo[0] * inv, hhi[0] + ahi[0] * inv], axis=1)
    out_r[...] = (
        jnp.dot(u, w2_r[...], preferred_element_type=jnp.float32) + b2_r[...])

  lo = pl.BlockSpec((1, _R, _HH), lambda i: (0, i, 0))
  hi = pl.BlockSpec((1, _R, _HH), lambda i: (1, i, 0))
  return pl.pallas_call(
      body,
      grid=(_N // _R,),
      in_specs=[
          lo, hi, lo, hi,
          pl.BlockSpec((_R, 1), lambda i: (i, 0)),
          _full((_H, 128)), _full((1, 128)),
      ],
      out_specs=pl.BlockSpec((_R, 128), lambda i: (i, 0)),
      out_shape=jax.ShapeDtypeStruct((_N, 128), jnp.float32),
  )(hp, hp, agg, agg, deg, w2, b2)


def kernel(x, c, edge_index, W1_0, b1_0, Wc_0, bc_0, W2_0, b2_0,
           W1_1, b1_1, Wc_1, bc_1, W2_1, b2_1):
  src = edge_index[0]
  dst = edge_index[1]
  pad = _EPAD - _E
  srcp = jnp.concatenate([src, jnp.zeros((pad,), jnp.int32)])
  dstp = jnp.concatenate([dst, jnp.full((pad,), _N, jnp.int32)])
  src3 = jnp.stack([srcp, srcp + _N]).reshape(2, _NROWS, _CHUNK)
  dst2 = dstp.reshape(_NROWS, _CHUNK)

  hp0, c1 = _tc_pre(x, c, W1_0[:128], W1_0[128:],
                    b1_0.reshape(1, _H), Wc_0, bc_0.reshape(1, _H))
  agg0, deg = _sc_agg_deg(hp0.reshape(2 * _N, _HH), src3, dst2)
  deg2 = deg.reshape(_N, 1)
  hp1 = _tc_mid(hp0, agg0, deg2, c1, W2_0, b2_0.reshape(1, _H),
                W1_1[:_H], W1_1[_H:], b1_1.reshape(1, _H))
  agg1, _unused_deg = _sc_agg_deg(hp1.reshape(2 * _N, _HH), src3, dst2)
  return _tc_fin(hp1, agg1, deg2, W2_1, b2_1.reshape(1, 128))
